# baseline (speedup 1.0000x reference)
import functools

import jax
import jax.numpy as jnp
from jax import lax
from jax.experimental import pallas as pl
from jax.experimental.pallas import tpu as pltpu
from jax.experimental.pallas import tpu_sc as plsc

NUM_CLS = 151
FILL_V = 1000.0
LANES = 16


def _sc_workers():
    try:
        info = plsc.get_sparse_core_info()
        return info.num_cores, info.num_subcores
    except Exception:
        return 2, 16


def _onehot_call(n_rows: int):
    NC, NS = _sc_workers()
    NW = NC * NS
    C = NUM_CLS
    R = -(-n_rows // NW)
    R = -(-R // LANES) * LANES
    chunk = R * C
    n_pad = R * NW
    w_pad = chunk * NW

    fill_iters = chunk // LANES
    unroll = 1
    for g in (32, 20, 16, 10, 8, 5, 4, 2):
        if fill_iters % g == 0:
            unroll = g
            break
    outer = fill_iters // unroll

    mesh = plsc.VectorSubcoreMesh(core_axis_name="c", subcore_axis_name="s")

    @functools.partial(
        pl.kernel,
        out_type=jax.ShapeDtypeStruct((w_pad,), jnp.float32),
        mesh=mesh,
        scratch_types=[
            pltpu.VMEM((R,), jnp.int32),
            pltpu.VMEM((chunk,), jnp.float32),
        ],
        compiler_params=pltpu.CompilerParams(needs_layout_passes=False),
    )
    def onehot_kernel(labels_hbm, out_hbm, lab_v, buf_v):
        wid = lax.axis_index("s") * NC + lax.axis_index("c")
        r0 = wid * R
        w0 = wid * chunk

        s_load = jnp.minimum(r0, n_rows - R)
        pltpu.sync_copy(labels_hbm.at[pl.ds(s_load, R)], lab_v)

        neg = jnp.full((LANES,), -FILL_V, dtype=jnp.float32)
        pos = jnp.full((LANES,), FILL_V, dtype=jnp.float32)
        lane = lax.iota(jnp.int32, LANES)

        def fill_body(k, _):
            base = k * (LANES * unroll)
            for t in range(unroll):
                buf_v[pl.ds(base + t * LANES, LANES)] = neg
            return _

        lax.fori_loop(0, outer, fill_body, None)

        for j in range(R // LANES):
            lab = lab_v[pl.ds(j * LANES, LANES)]
            r = s_load + (j * LANES) + lane
            local = (r - r0) * C + lab
            valid = (r >= r0) & (r < n_rows)
            idx = jnp.minimum(jnp.maximum(local, 0), chunk - 1)
            plsc.store_scatter(buf_v, [idx], pos, mask=valid)

        pltpu.sync_copy(buf_v, out_hbm.at[pl.ds(w0, chunk)])

    return onehot_kernel, n_pad, w_pad


def kernel(im_inds, obj_fmaps, obj_labels, rel_inds):
    n = obj_labels.shape[0]
    call, n_pad, w_pad = _onehot_call(n)
    flat = call(obj_labels)
    return flat[: n * NUM_CLS].reshape(n, NUM_CLS)


# speedup vs baseline: 1.0539x; 1.0539x over previous
import functools

import jax
import jax.numpy as jnp
from jax import lax
from jax.experimental import pallas as pl
from jax.experimental.pallas import tpu as pltpu
from jax.experimental.pallas import tpu_sc as plsc

NUM_CLS = 151
FILL_V = 1000.0
LANES = 16


def _sc_workers():
    try:
        info = plsc.get_sparse_core_info()
        return info.num_cores, info.num_subcores
    except Exception:
        return 2, 16


def _onehot_call(n_rows: int):
    NC, NS = _sc_workers()
    NW = NC * NS
    C = NUM_CLS
    assert n_rows % 8 == 0
    lo = (n_rows // NW) // 8 * 8
    hi = lo + 8
    n_hi = (n_rows - lo * NW) // 8
    assert n_hi * hi + (NW - n_hi) * lo == n_rows and 0 <= n_hi <= NW

    chunk_hi = hi * C
    chunk_lo = lo * C
    lab_win = hi

    fill_iters = -(-chunk_hi // LANES)
    unroll = 1
    for g in (32, 20, 16, 10, 8, 5, 4, 2):
        if fill_iters % g == 0:
            unroll = g
            break
    outer = fill_iters // unroll

    mesh = plsc.VectorSubcoreMesh(core_axis_name="c", subcore_axis_name="s")

    @functools.partial(
        pl.kernel,
        out_type=jax.ShapeDtypeStruct((n_rows * C,), jnp.float32),
        mesh=mesh,
        scratch_types=[
            pltpu.VMEM((lab_win,), jnp.int32),
            pltpu.VMEM((outer * unroll * LANES,), jnp.float32),
        ],
        compiler_params=pltpu.CompilerParams(needs_layout_passes=False),
    )
    def onehot_kernel(labels_hbm, out_hbm, lab_v, buf_v):
        wid = lax.axis_index("s") * NC + lax.axis_index("c")
        in_hi = wid < n_hi
        r0 = jnp.where(in_hi, wid * hi, n_hi * hi + (wid - n_hi) * lo)
        my_rows = jnp.where(in_hi, hi, lo)
        w0 = r0 * C

        s_load = jnp.minimum(r0, n_rows - lab_win)
        pltpu.sync_copy(labels_hbm.at[pl.ds(s_load, lab_win)], lab_v)

        neg = jnp.full((LANES,), -FILL_V, dtype=jnp.float32)
        pos = jnp.full((LANES,), FILL_V, dtype=jnp.float32)
        lane = lax.iota(jnp.int32, LANES)

        def fill_body(k, _):
            base = k * (LANES * unroll)
            for t in range(unroll):
                buf_v[pl.ds(base + t * LANES, LANES)] = neg
            return _

        lax.fori_loop(0, outer, fill_body, None)

        for j in range(lab_win // LANES):
            lab = lab_v[pl.ds(j * LANES, LANES)]
            r = s_load + (j * LANES) + lane
            local = (r - r0) * C + lab
            valid = (r >= r0) & (r < r0 + my_rows)
            idx = jnp.minimum(jnp.maximum(local, 0), chunk_hi - 1)
            plsc.store_scatter(buf_v, [idx], pos, mask=valid)

        @pl.when(in_hi)
        def _():
            pltpu.sync_copy(buf_v.at[pl.ds(0, chunk_hi)],
                            out_hbm.at[pl.ds(w0, chunk_hi)])

        @pl.when(jnp.logical_not(in_hi))
        def _():
            pltpu.sync_copy(buf_v.at[pl.ds(0, chunk_lo)],
                            out_hbm.at[pl.ds(w0, chunk_lo)])

    return onehot_kernel


def kernel(im_inds, obj_fmaps, obj_labels, rel_inds):
    n = obj_labels.shape[0]
    call = _onehot_call(n)
    flat = call(obj_labels)
    return flat.reshape(n, NUM_CLS)


# speedup vs baseline: 2.5164x; 2.3877x over previous
import functools

import jax
import jax.numpy as jnp
from jax import lax
from jax.experimental import pallas as pl
from jax.experimental.pallas import tpu as pltpu
from jax.experimental.pallas import tpu_sc as plsc

NUM_CLS = 151
FILL_V = 1000.0
LANES = 16


def _sc_workers():
    try:
        info = plsc.get_sparse_core_info()
        return info.num_cores, info.num_subcores
    except Exception:
        return 2, 16


def _onehot_call(n_rows: int):
    NC, NS = _sc_workers()
    NW = NC * NS
    C = NUM_CLS
    assert n_rows % 8 == 0
    lo = (n_rows // NW) // 8 * 8
    hi = lo + 8
    n_hi = (n_rows - lo * NW) // 8
    assert n_hi * hi + (NW - n_hi) * lo == n_rows and 0 <= n_hi <= NW

    col_offs = list(range(0, C - LANES + 1, LANES))
    if col_offs[-1] + LANES < C:
        last = C - LANES
        t_lo = col_offs[-1] + LANES
        assert last // 128 == (C - 1) // 128 and last >= 0 and t_lo // 128 == last // 128
        col_offs.append(last)

    mesh = plsc.VectorSubcoreMesh(core_axis_name="c", subcore_axis_name="s")

    @functools.partial(
        pl.kernel,
        out_type=jax.ShapeDtypeStruct((n_rows, C), jnp.float32),
        mesh=mesh,
        scratch_types=[
            pltpu.VMEM((hi,), jnp.int32),
            pltpu.VMEM((hi, C), jnp.float32),
        ],
        compiler_params=pltpu.CompilerParams(needs_layout_passes=False),
    )
    def onehot_kernel(labels_hbm, out_hbm, lab_v, buf_v):
        wid = lax.axis_index("s") * NC + lax.axis_index("c")
        in_hi = wid < n_hi
        r0 = jnp.where(in_hi, wid * hi, n_hi * hi + (wid - n_hi) * lo)
        my_rows = jnp.where(in_hi, hi, lo)

        s_load = jnp.minimum(r0, n_rows - hi)
        pltpu.sync_copy(labels_hbm.at[pl.ds(s_load, hi)], lab_v)

        neg = jnp.full((LANES,), -FILL_V, dtype=jnp.float32)
        pos = jnp.full((LANES,), FILL_V, dtype=jnp.float32)
        lane = lax.iota(jnp.int32, LANES)

        def fill_body(i, _):
            for o in col_offs:
                buf_v[i, pl.ds(o, LANES)] = neg
            return _

        lax.fori_loop(0, hi, fill_body, None)

        for j in range(hi // LANES):
            lab = lab_v[pl.ds(j * LANES, LANES)]
            r = s_load + (j * LANES) + lane
            rl = r - r0
            valid = (rl >= 0) & (rl < my_rows)
            rl_c = jnp.minimum(jnp.maximum(rl, 0), hi - 1)
            plsc.store_scatter(buf_v, [rl_c, lab], pos, mask=valid)

        @pl.when(in_hi)
        def _():
            pltpu.sync_copy(buf_v.at[pl.ds(0, hi)],
                            out_hbm.at[pl.ds(r0, hi)])

        @pl.when(jnp.logical_not(in_hi))
        def _():
            pltpu.sync_copy(buf_v.at[pl.ds(0, lo)],
                            out_hbm.at[pl.ds(r0, lo)])

    return onehot_kernel


def kernel(im_inds, obj_fmaps, obj_labels, rel_inds):
    n = obj_labels.shape[0]
    call = _onehot_call(n)
    return call(obj_labels)


# speedup vs baseline: 2.9939x; 1.1897x over previous
import functools

import jax
import jax.numpy as jnp
from jax import lax
from jax.experimental import pallas as pl
from jax.experimental.pallas import tpu as pltpu
from jax.experimental.pallas import tpu_sc as plsc

NUM_CLS = 151
FILL_V = 1000.0
LANES = 16
STRIP = 128


def _sc_workers():
    try:
        info = plsc.get_sparse_core_info()
        return info.num_cores, info.num_subcores
    except Exception:
        return 2, 16


def _onehot_call(n_rows: int):
    NC, NS = _sc_workers()
    NW = NC * NS
    C = NUM_CLS
    assert n_rows % 8 == 0 and n_rows >= STRIP
    c_pad = -(-C // 8) * 8
    n_tiles = c_pad // 8
    n_strips = -(-n_rows // STRIP)
    max_strips = -(-n_strips // NW)

    mesh = plsc.VectorSubcoreMesh(core_axis_name="c", subcore_axis_name="s")

    @functools.partial(
        pl.kernel,
        out_type=jax.ShapeDtypeStruct((C, n_rows), jnp.float32),
        mesh=mesh,
        scratch_types=[
            pltpu.VMEM((STRIP,), jnp.int32),
            pltpu.VMEM((c_pad, STRIP), jnp.float32),
        ],
        compiler_params=pltpu.CompilerParams(needs_layout_passes=False),
    )
    def onehot_kernel(labels_hbm, out_hbm, lab_v, buf_v):
        wid = lax.axis_index("s") * NC + lax.axis_index("c")

        neg = jnp.full((LANES,), -FILL_V, dtype=jnp.float32)
        pos = jnp.full((LANES,), FILL_V, dtype=jnp.float32)
        lane = lax.iota(jnp.int32, LANES)

        def fill_body(rr, _):
            for o in range(0, STRIP, LANES):
                buf_v[rr, pl.ds(o, LANES)] = neg
            return _

        lax.fori_loop(0, c_pad, fill_body, None)

        for k in range(max_strips):
            s = wid + k * NW

            @pl.when(s < n_strips)
            def _():
                i0 = s * STRIP
                s_load = jnp.minimum(i0, n_rows - STRIP)
                pltpu.sync_copy(labels_hbm.at[pl.ds(s_load, STRIP)], lab_v)

                chunks = []
                for j in range(STRIP // LANES):
                    lab = lab_v[pl.ds(j * LANES, LANES)]
                    i = s_load + (j * LANES) + lane
                    il = i - i0
                    valid = (il >= 0) & (il < STRIP) & (i < n_rows)
                    il_c = jnp.minimum(jnp.maximum(il, 0), STRIP - 1)
                    chunks.append((lab, il_c, valid))
                    plsc.store_scatter(buf_v, [lab, il_c], pos, mask=valid)

                last_w = n_rows - (n_strips - 1) * STRIP

                @pl.when(s < n_strips - 1)
                def _():
                    for tr in range(n_tiles):
                        rows = min(8, C - tr * 8)
                        pltpu.sync_copy(
                            buf_v.at[pl.ds(tr * 8, rows)],
                            out_hbm.at[pl.ds(tr * 8, rows), pl.ds(i0, STRIP)],
                        )

                @pl.when(s == n_strips - 1)
                def _():
                    for tr in range(n_tiles):
                        rows = min(8, C - tr * 8)
                        pltpu.sync_copy(
                            buf_v.at[pl.ds(tr * 8, rows), pl.ds(0, last_w)],
                            out_hbm.at[pl.ds(tr * 8, rows), pl.ds(i0, last_w)],
                        )

                for lab, il_c, valid in chunks:
                    plsc.store_scatter(buf_v, [lab, il_c], neg, mask=valid)

    return onehot_kernel


def kernel(im_inds, obj_fmaps, obj_labels, rel_inds):
    n = obj_labels.shape[0]
    call = _onehot_call(n)
    return call(obj_labels).T


# speedup vs baseline: 3.3276x; 1.1115x over previous
import functools

import jax
import jax.numpy as jnp
from jax import lax
from jax.experimental import pallas as pl
from jax.experimental.pallas import tpu as pltpu
from jax.experimental.pallas import tpu_sc as plsc

NUM_CLS = 151
FILL_V = 1000.0
LANES = 16
STRIP = 128


def _sc_workers():
    try:
        info = plsc.get_sparse_core_info()
        return info.num_cores, info.num_subcores
    except Exception:
        return 2, 16


def _onehot_call(n_rows: int):
    NC, NS = _sc_workers()
    NW = NC * NS
    C = NUM_CLS
    assert n_rows % 8 == 0 and n_rows >= STRIP
    c_pad = -(-C // 8) * 8
    n_tiles = c_pad // 8
    n_strips = -(-n_rows // STRIP)
    max_strips = -(-n_strips // NW)

    mesh = plsc.VectorSubcoreMesh(core_axis_name="c", subcore_axis_name="s")

    @functools.partial(
        pl.kernel,
        out_type=jax.ShapeDtypeStruct((C, n_rows), jnp.float32),
        mesh=mesh,
        scratch_types=[
            pltpu.VMEM((STRIP,), jnp.int32),
            pltpu.VMEM((c_pad, STRIP), jnp.float32),
            pltpu.SemaphoreType.DMA,
        ],
        compiler_params=pltpu.CompilerParams(needs_layout_passes=False),
    )
    def onehot_kernel(labels_hbm, out_hbm, lab_v, buf_v, sem):
        wid = lax.axis_index("s") * NC + lax.axis_index("c")

        neg = jnp.full((LANES,), -FILL_V, dtype=jnp.float32)
        pos = jnp.full((LANES,), FILL_V, dtype=jnp.float32)
        lane = lax.iota(jnp.int32, LANES)

        def fill_body(rr, _):
            for o in range(0, STRIP, LANES):
                buf_v[rr, pl.ds(o, LANES)] = neg
            return _

        lax.fori_loop(0, c_pad, fill_body, None)

        for k in range(max_strips):
            s = wid + k * NW

            @pl.when(s < n_strips)
            def _():
                i0 = s * STRIP
                s_load = jnp.minimum(i0, n_rows - STRIP)
                pltpu.sync_copy(labels_hbm.at[pl.ds(s_load, STRIP)], lab_v)

                chunks = []
                for j in range(STRIP // LANES):
                    lab = lab_v[pl.ds(j * LANES, LANES)]
                    i = s_load + (j * LANES) + lane
                    il = i - i0
                    valid = (il >= 0) & (il < STRIP) & (i < n_rows)
                    il_c = jnp.minimum(jnp.maximum(il, 0), STRIP - 1)
                    chunks.append((lab, il_c, valid))
                    plsc.store_scatter(buf_v, [lab, il_c], pos, mask=valid)

                last_w = n_rows - (n_strips - 1) * STRIP

                @pl.when(s < n_strips - 1)
                def _():
                    handles = []
                    for tr in range(n_tiles):
                        rows = min(8, C - tr * 8)
                        handles.append(pltpu.async_copy(
                            buf_v.at[pl.ds(tr * 8, rows)],
                            out_hbm.at[pl.ds(tr * 8, rows), pl.ds(i0, STRIP)],
                            sem,
                        ))
                    for h in handles:
                        h.wait()

                @pl.when(s == n_strips - 1)
                def _():
                    handles = []
                    for tr in range(n_tiles):
                        rows = min(8, C - tr * 8)
                        handles.append(pltpu.async_copy(
                            buf_v.at[pl.ds(tr * 8, rows), pl.ds(0, last_w)],
                            out_hbm.at[pl.ds(tr * 8, rows), pl.ds(i0, last_w)],
                            sem,
                        ))
                    for h in handles:
                        h.wait()

                for lab, il_c, valid in chunks:
                    plsc.store_scatter(buf_v, [lab, il_c], neg, mask=valid)

    return onehot_kernel


def kernel(im_inds, obj_fmaps, obj_labels, rel_inds):
    n = obj_labels.shape[0]
    call = _onehot_call(n)
    return call(obj_labels).T


# speedup vs baseline: 3.4426x; 1.0346x over previous
import functools

import jax
import jax.numpy as jnp
from jax import lax
from jax.experimental import pallas as pl
from jax.experimental.pallas import tpu as pltpu
from jax.experimental.pallas import tpu_sc as plsc

NUM_CLS = 151
FILL_V = 1000.0
LANES = 16
STRIP = 128


def _sc_workers():
    try:
        info = plsc.get_sparse_core_info()
        return info.num_cores, info.num_subcores
    except Exception:
        return 2, 16


def _onehot_call(n_rows: int):
    NC, NS = _sc_workers()
    NW = NC * NS
    C = NUM_CLS
    assert n_rows % 8 == 0 and n_rows >= STRIP
    c_pad = -(-C // 8) * 8
    n_tiles = c_pad // 8
    n_strips = -(-n_rows // STRIP)
    max_strips = -(-n_strips // NW)
    kF = max_strips - 1
    last_w = n_rows - (n_strips - 1) * STRIP
    i0_last = (n_strips - 1) * STRIP
    shift = i0_last - (n_rows - STRIP)

    mesh = plsc.VectorSubcoreMesh(core_axis_name="c", subcore_axis_name="s")

    scratch = [pltpu.VMEM((STRIP,), jnp.int32) for _ in range(max_strips)]
    scratch += [pltpu.VMEM((c_pad, STRIP), jnp.float32) for _ in range(2)]
    scratch += [pltpu.SemaphoreType.DMA for _ in range(max_strips + 2)]

    @functools.partial(
        pl.kernel,
        out_type=jax.ShapeDtypeStruct((C, n_rows), jnp.float32),
        mesh=mesh,
        scratch_types=scratch,
        compiler_params=pltpu.CompilerParams(needs_layout_passes=False),
    )
    def onehot_kernel(labels_hbm, out_hbm, *scr):
        labs = scr[:max_strips]
        bufs = scr[max_strips:max_strips + 2]
        sem_labs = scr[max_strips + 2:2 * max_strips + 2]
        sems = scr[2 * max_strips + 2:2 * max_strips + 4]
        wid = lax.axis_index("s") * NC + lax.axis_index("c")

        neg = jnp.full((LANES,), -FILL_V, dtype=jnp.float32)
        pos = jnp.full((LANES,), FILL_V, dtype=jnp.float32)
        lane = lax.iota(jnp.int32, LANES)

        lab_descs = []
        for k in range(max_strips):
            s = wid + k * NW
            i0_lab = jnp.minimum(s * STRIP, n_rows - STRIP)
            d = pltpu.make_async_copy(
                labels_hbm.at[pl.ds(i0_lab, STRIP)], labs[k], sem_labs[k])
            d.start()
            lab_descs.append(d)

        def fill_body(rr, _):
            for b in range(2):
                for o in range(0, STRIP, LANES):
                    bufs[b][rr, pl.ds(o, LANES)] = neg
            return _

        lax.fori_loop(0, c_pad, fill_body, None)

        for d in lab_descs:
            d.wait()

        def scatter(buf, labref, val):
            for j in range(STRIP // LANES):
                lab = labref[pl.ds(j * LANES, LANES)]
                plsc.store_scatter(buf, [lab, (j * LANES) + lane], val)

        def fire(buf, i0, sem):
            descs = []
            for tr in range(n_tiles):
                rows = min(8, C - tr * 8)
                d = pltpu.make_async_copy(
                    buf.at[pl.ds(tr * 8, rows)],
                    out_hbm.at[pl.ds(tr * 8, rows), pl.ds(i0, STRIP)],
                    sem,
                )
                d.start()
                descs.append(d)
            return descs

        out_descs = []
        for k in range(kF):
            b = k % 2
            if k >= 2:
                for d in out_descs[k - 2]:
                    d.wait()
                scatter(bufs[b], labs[k - 2], neg)
            scatter(bufs[b], labs[k], pos)
            out_descs.append(fire(bufs[b], (wid + k * NW) * STRIP, sems[b]))

        bF = kF % 2
        sF = wid + kF * NW
        if kF >= 2:
            for d in out_descs[kF - 2]:
                d.wait()

            @pl.when(sF < n_strips)
            def _():
                scatter(bufs[bF], labs[kF - 2], neg)

        @pl.when(sF < n_strips - 1)
        def _():
            scatter(bufs[bF], labs[kF], pos)
            for d in fire(bufs[bF], sF * STRIP, sems[bF]):
                d.wait()

        @pl.when(sF == n_strips - 1)
        def _():
            for j in range(STRIP // LANES):
                lab = labs[kF][pl.ds(j * LANES, LANES)]
                il = (j * LANES) + lane - shift
                plsc.store_scatter(bufs[bF], [lab, jnp.maximum(il, 0)], pos,
                                   mask=il >= 0)
            i0p = sF * STRIP
            for tr in range(n_tiles):
                rows = min(8, C - tr * 8)
                pltpu.sync_copy(
                    bufs[bF].at[pl.ds(tr * 8, rows), pl.ds(0, last_w)],
                    out_hbm.at[pl.ds(tr * 8, rows), pl.ds(i0p, last_w)],
                )

        for k in range(max(0, kF - 1), kF):
            for d in out_descs[k]:
                d.wait()

    return onehot_kernel


def kernel(im_inds, obj_fmaps, obj_labels, rel_inds):
    n = obj_labels.shape[0]
    call = _onehot_call(n)
    return call(obj_labels).T
